# baseline (speedup 1.0000x reference)
import functools

import jax
import jax.numpy as jnp
from jax import lax
from jax.experimental import pallas as pl
from jax.experimental.pallas import tpu as pltpu
from jax.experimental.pallas import tpu_sc as plsc

_LANES = 16


def kernel(input, table):
    B, L = input.shape
    V = table.shape[0]
    info = plsc.get_sparse_core_info()
    nw = info.num_cores * info.num_subcores
    rows_w = B // nw
    per_w = rows_w * L
    half_rows = rows_w // 2
    half = half_rows * L
    assert rows_w * nw == B and half % _LANES == 0

    mesh = plsc.VectorSubcoreMesh(core_axis_name="c", subcore_axis_name="s")

    @functools.partial(
        pl.kernel,
        mesh=mesh,
        compiler_params=pltpu.CompilerParams(
            needs_layout_passes=False, use_tc_tiling_on_sc=True),
        out_type=jax.ShapeDtypeStruct((B, L), jnp.float32),
        scratch_types=[
            pltpu.VMEM((rows_w, L), jnp.int32),
            pltpu.VMEM((rows_w, L), jnp.float32),
            pltpu.VMEM((V,), jnp.float32),
            pltpu.SemaphoreType.DMA,
            pltpu.SemaphoreType.DMA,
            pltpu.SemaphoreType.DMA,
            pltpu.SemaphoreType.DMA,
        ],
    )
    def lookup(idx_hbm, tab_hbm, out_hbm, idx_v, out_v, tab_v, si0, si1, so0, so1):
        wid = lax.axis_index("s") * info.num_cores + lax.axis_index("c")
        r0 = wid * rows_w
        in0 = pltpu.async_copy(
            idx_hbm.at[pl.ds(r0, half_rows)], idx_v.at[pl.ds(0, half_rows)], si0)
        in1 = pltpu.async_copy(
            idx_hbm.at[pl.ds(r0 + half_rows, half_rows)],
            idx_v.at[pl.ds(half_rows, half_rows)], si1)
        pltpu.sync_copy(tab_hbm, tab_v)
        col_starts = [k * _LANES for k in range(L // _LANES)] + [L - _LANES]

        def gather_row(r):
            for c0 in col_starts:
                iv = idx_v[r, pl.ds(c0, _LANES)]
                out_v[r, pl.ds(c0, _LANES)] = plsc.load_gather(tab_v, [iv])

        in0.wait()

        @plsc.parallel_loop(0, half_rows, step=1, unroll=2)
        def body0(r):
            gather_row(r)

        out0 = pltpu.async_copy(
            out_v.at[pl.ds(0, half_rows)], out_hbm.at[pl.ds(r0, half_rows)], so0)
        in1.wait()

        @plsc.parallel_loop(half_rows, rows_w, step=1, unroll=2)
        def body1(r):
            gather_row(r)

        out1 = pltpu.async_copy(
            out_v.at[pl.ds(half_rows, half_rows)],
            out_hbm.at[pl.ds(r0 + half_rows, half_rows)], so1)
        out0.wait()
        out1.wait()

    return lookup(input, table.reshape(V))


# speedup vs baseline: 1.2905x; 1.2905x over previous
import functools

import jax
import jax.numpy as jnp
from jax import lax
from jax.experimental import pallas as pl
from jax.experimental.pallas import tpu as pltpu
from jax.experimental.pallas import tpu_sc as plsc

_LANES = 16
_CH = 4


def kernel(input, table):
    B, L = input.shape
    V = table.shape[0]
    xt = input.T
    nch = L // _CH
    info = plsc.get_sparse_core_info()
    nw = info.num_cores * info.num_subcores
    assert nch * _CH == L and B % (8 * _LANES) == 0

    mesh = plsc.VectorSubcoreMesh(core_axis_name="c", subcore_axis_name="s")

    @functools.partial(
        pl.kernel,
        mesh=mesh,
        compiler_params=pltpu.CompilerParams(needs_layout_passes=False),
        out_type=jax.ShapeDtypeStruct((L, B), jnp.float32),
        scratch_types=[
            pltpu.VMEM((2 * _CH, B), jnp.int32),
            pltpu.VMEM((2 * _CH, B), jnp.float32),
            pltpu.VMEM((V,), jnp.float32),
            pltpu.SemaphoreType.DMA,
            pltpu.SemaphoreType.DMA,
            pltpu.SemaphoreType.DMA,
            pltpu.SemaphoreType.DMA,
        ],
    )
    def lookup(idx_hbm, tab_hbm, out_hbm, idx_v, out_v, tab_v, sia, sib, soa, sob):
        wid = lax.axis_index("s") * info.num_cores + lax.axis_index("c")
        ra = wid * _CH
        rb = lax.rem(ra + nw * _CH, nch * _CH)
        ina = pltpu.async_copy(
            idx_hbm.at[pl.ds(ra, _CH)], idx_v.at[pl.ds(0, _CH)], sia)
        inb = pltpu.async_copy(
            idx_hbm.at[pl.ds(rb, _CH)], idx_v.at[pl.ds(_CH, _CH)], sib)
        pltpu.sync_copy(tab_hbm, tab_v)

        def gather_chunk(s0):
            @plsc.parallel_loop(0, B, step=_LANES, unroll=4)
            def body(off):
                for s in range(s0, s0 + _CH):
                    iv = idx_v[s, pl.ds(off, _LANES)]
                    out_v[s, pl.ds(off, _LANES)] = plsc.load_gather(tab_v, [iv])

        ina.wait()
        gather_chunk(0)
        outa = pltpu.async_copy(
            out_v.at[pl.ds(0, _CH)], out_hbm.at[pl.ds(ra, _CH)], soa)
        inb.wait()
        gather_chunk(_CH)
        outb = pltpu.async_copy(
            out_v.at[pl.ds(_CH, _CH)], out_hbm.at[pl.ds(rb, _CH)], sob)
        outa.wait()
        outb.wait()

    return lookup(xt, table.reshape(V)).T


# speedup vs baseline: 1.3353x; 1.0347x over previous
import functools

import jax
import jax.numpy as jnp
from jax import lax
from jax.experimental import pallas as pl
from jax.experimental.pallas import tpu as pltpu
from jax.experimental.pallas import tpu_sc as plsc

_LANES = 16
_SLAB = 128


def kernel(input, table):
    B, L = input.shape
    V = table.shape[0]
    xt = input.T
    info = plsc.get_sparse_core_info()
    nw = info.num_cores * info.num_subcores
    half = (L // 2) // 8 * 8
    rest = L - half
    assert nw * _SLAB == B and half % 8 == 0 and rest % 8 == 0

    mesh = plsc.VectorSubcoreMesh(core_axis_name="c", subcore_axis_name="s")

    @functools.partial(
        pl.kernel,
        mesh=mesh,
        compiler_params=pltpu.CompilerParams(needs_layout_passes=False),
        out_type=jax.ShapeDtypeStruct((L, B), jnp.float32),
        scratch_types=[
            pltpu.VMEM((L, _SLAB), jnp.int32),
            pltpu.VMEM((L, _SLAB), jnp.float32),
            pltpu.VMEM((V,), jnp.float32),
            pltpu.SemaphoreType.DMA,
            pltpu.SemaphoreType.DMA,
            pltpu.SemaphoreType.DMA,
            pltpu.SemaphoreType.DMA,
        ],
    )
    def lookup(idx_hbm, tab_hbm, out_hbm, idx_v, out_v, tab_v, sia, sib, soa, sob):
        wid = lax.axis_index("s") * info.num_cores + lax.axis_index("c")
        c0 = wid * _SLAB
        ina = pltpu.async_copy(
            idx_hbm.at[pl.ds(0, half), pl.ds(c0, _SLAB)],
            idx_v.at[pl.ds(0, half)], sia)
        inb = pltpu.async_copy(
            idx_hbm.at[pl.ds(half, rest), pl.ds(c0, _SLAB)],
            idx_v.at[pl.ds(half, rest)], sib)
        pltpu.sync_copy(tab_hbm, tab_v)

        def gather_half(r0, n):
            @plsc.parallel_loop(r0, r0 + n, step=1, unroll=2)
            def body(r):
                for k in range(_SLAB // _LANES):
                    iv = idx_v[r, pl.ds(k * _LANES, _LANES)]
                    out_v[r, pl.ds(k * _LANES, _LANES)] = plsc.load_gather(
                        tab_v, [iv])

        ina.wait()
        gather_half(0, half)
        outa = pltpu.async_copy(
            out_v.at[pl.ds(0, half)],
            out_hbm.at[pl.ds(0, half), pl.ds(c0, _SLAB)], soa)
        inb.wait()
        gather_half(half, rest)
        outb = pltpu.async_copy(
            out_v.at[pl.ds(half, rest)],
            out_hbm.at[pl.ds(half, rest), pl.ds(c0, _SLAB)], sob)
        outa.wait()
        outb.wait()

    return lookup(xt, table.reshape(V)).T
